# baseline (speedup 1.0000x reference)
import jax
import jax.numpy as jnp
from jax import lax
from jax.experimental import pallas as pl
from jax.experimental.pallas import tpu as pltpu
from jax.experimental.pallas import tpu_sc as plsc

_M, _U, _V = 8, 400, 400
_T = _M * _U * _V
_NW = 32
_C = 1024


def _adapter_body(x_hbm, m_hbm, u_hbm, v_hbm, ct_hbm, out_hbm,
                  u_v, v_v, m_v, x_v,
                  i11, i21, i12, i22,
                  w11, w21, w12, w22,
                  r11, r21, r12, r22,
                  out_v, sem):
    wid = lax.axis_index("s") * 2 + lax.axis_index("c")
    n_total = m_hbm.shape[0]
    per_w = n_total // _NW
    n_chunks = per_w // _C
    iota = lax.iota(jnp.int32, 16)

    def chunk_body(c, carry):
        off = wid * per_w + c * _C
        in_copies = [
            pltpu.async_copy(u_hbm.at[pl.ds(off, _C)], u_v, sem),
            pltpu.async_copy(v_hbm.at[pl.ds(off, _C)], v_v, sem),
            pltpu.async_copy(m_hbm.at[pl.ds(off, _C)], m_v, sem),
            pltpu.async_copy(x_hbm.at[pl.ds(3 * off, 3 * _C)], x_v, sem),
        ]
        for cp in in_copies:
            cp.wait()

        def prep(g, carry2):
            s = pl.multiple_of(g * 16, 16)
            u16 = u_v[pl.ds(s, 16)]
            v16 = v_v[pl.ds(s, 16)]
            m16 = m_v[pl.ds(s, 16)]
            iu = u16 * jnp.float32(_U)
            iu = jnp.where(iu == jnp.float32(_U), jnp.float32(_U - 1), iu)
            iv = v16 * jnp.float32(_V)
            iv = jnp.where(iv == jnp.float32(_V), jnp.float32(_V - 1), iv)
            i1 = iu.astype(jnp.int32)
            j1 = iv.astype(jnp.int32)
            ir = iu - i1.astype(jnp.float32)
            jr = iv - j1.astype(jnp.float32)
            i2 = jnp.where(i1 == _U - 1, 0, i1 + 1)
            j2 = jnp.where(j1 == _V - 1, 0, j1 + 1)
            base = m16 * (_U * _V)
            row1 = base + i1 * _V
            row2 = base + i2 * _V
            i11[pl.ds(s, 16)] = row1 + j1
            i21[pl.ds(s, 16)] = row2 + j1
            i12[pl.ds(s, 16)] = row1 + j2
            i22[pl.ds(s, 16)] = row2 + j2
            omi = 1.0 - ir
            omj = 1.0 - jr
            w11[pl.ds(s, 16)] = omi * omj
            w21[pl.ds(s, 16)] = ir * omj
            w12[pl.ds(s, 16)] = omi * jr
            w22[pl.ds(s, 16)] = ir * jr
            return carry2
        lax.fori_loop(0, _C // 16, prep, 0)

        copies = [pltpu.async_copy(ct_hbm.at[idx_v], rm_v, sem)
                  for idx_v, rm_v in ((i11, r11), (i21, r21),
                                      (i12, r12), (i22, r22))]
        for cp in copies:
            cp.wait()

        def comb(g, carry2):
            s = pl.multiple_of(g * 16, 16)
            rows = iota + s
            r3 = rows * 3
            wa = w11[pl.ds(s, 16)]
            wb = w21[pl.ds(s, 16)]
            wc = w12[pl.ds(s, 16)]
            wd = w22[pl.ds(s, 16)]

            def gat(ref, k):
                return plsc.load_gather(ref, [rows, jnp.full((16,), k, jnp.int32)])

            A = [wa * gat(r11, k) + wb * gat(r21, k)
                 + wc * gat(r12, k) + wd * gat(r22, k) for k in range(12)]
            x0 = plsc.load_gather(x_v, [r3])
            x1 = plsc.load_gather(x_v, [r3 + 1])
            x2 = plsc.load_gather(x_v, [r3 + 2])
            for j in range(3):
                yj = x0 * A[j] + x1 * A[3 + j] + x2 * A[6 + j] + A[9 + j]
                plsc.store_scatter(out_v, [r3 + j], yj)
            return carry2
        lax.fori_loop(0, _C // 16, comb, 0)

        pltpu.sync_copy(out_v, out_hbm.at[pl.ds(3 * off, 3 * _C)])
        return carry
    lax.fori_loop(0, n_chunks, chunk_body, 0)


_CB = 2000


def _pack_body(mt_hbm, bt_hbm, ct_hbm, mt_v, bt_v, out_v, sem):
    wid = lax.axis_index("s") * 2 + lax.axis_index("c")
    per_w = _T // _NW
    iota = lax.iota(jnp.int32, 16)

    def chunk_body(c, carry):
        off = wid * per_w + c * _CB
        in_copies = [
            pltpu.async_copy(mt_hbm.at[pl.ds(9 * off, 9 * _CB)], mt_v, sem),
            pltpu.async_copy(bt_hbm.at[pl.ds(3 * off, 3 * _CB)], bt_v, sem),
        ]
        for cp in in_copies:
            cp.wait()

        def interleave(g, carry2):
            rec = iota + pl.multiple_of(g * 16, 16)
            r9 = rec * 9
            r3 = rec * 3
            for k in range(9):
                plsc.store_scatter(
                    out_v, [rec, jnp.full((16,), k, jnp.int32)],
                    plsc.load_gather(mt_v, [r9 + k]))
            for k in range(3):
                plsc.store_scatter(
                    out_v, [rec, jnp.full((16,), 9 + k, jnp.int32)],
                    plsc.load_gather(bt_v, [r3 + k]))
            return carry2
        lax.fori_loop(0, _CB // 16, interleave, 0)

        pltpu.sync_copy(out_v, ct_hbm.at[pl.ds(off, _CB)])
        return carry
    lax.fori_loop(0, per_w // _CB, chunk_body, 0)


def _pack_tables(m_param, b_param):
    f32 = jnp.float32
    mesh = plsc.VectorSubcoreMesh(core_axis_name="c", subcore_axis_name="s")
    f = pl.kernel(
        _pack_body,
        mesh=mesh,
        out_type=jax.ShapeDtypeStruct((_T, 16), f32),
        compiler_params=pltpu.CompilerParams(
            needs_layout_passes=False, use_tc_tiling_on_sc=False),
        scratch_types=[
            pltpu.VMEM((_CB * 9,), f32),
            pltpu.VMEM((_CB * 3,), f32),
            pltpu.VMEM((_CB, 16), f32),
            pltpu.SemaphoreType.DMA,
        ],
    )
    return f(m_param.reshape(-1), b_param.reshape(-1))


def kernel(x, m, u, v, m_param, b_param):
    n = x.shape[0]
    xf = x.reshape(-1)
    ct = _pack_tables(m_param, b_param)
    f32, i32 = jnp.float32, jnp.int32
    mesh = plsc.VectorSubcoreMesh(core_axis_name="c", subcore_axis_name="s")
    f = pl.kernel(
        _adapter_body,
        mesh=mesh,
        out_type=jax.ShapeDtypeStruct((n * 3,), f32),
        compiler_params=pltpu.CompilerParams(
            needs_layout_passes=False, use_tc_tiling_on_sc=False),
        scratch_types=[
            pltpu.VMEM((_C,), f32),
            pltpu.VMEM((_C,), f32),
            pltpu.VMEM((_C,), i32),
            pltpu.VMEM((_C * 3,), f32),
            pltpu.VMEM((_C,), i32),
            pltpu.VMEM((_C,), i32),
            pltpu.VMEM((_C,), i32),
            pltpu.VMEM((_C,), i32),
            pltpu.VMEM((_C,), f32),
            pltpu.VMEM((_C,), f32),
            pltpu.VMEM((_C,), f32),
            pltpu.VMEM((_C,), f32),
            pltpu.VMEM((_C, 16), f32),
            pltpu.VMEM((_C, 16), f32),
            pltpu.VMEM((_C, 16), f32),
            pltpu.VMEM((_C, 16), f32),
            pltpu.VMEM((_C * 3,), f32),
            pltpu.SemaphoreType.DMA,
        ],
    )
    return f(xf, m, u, v, ct).reshape(n, 3)


# speedup vs baseline: 5.6747x; 5.6747x over previous
import jax
import jax.numpy as jnp
from jax import lax
from jax.experimental import pallas as pl
from jax.experimental.pallas import tpu as pltpu
from jax.experimental.pallas import tpu_sc as plsc

_M, _U, _V = 8, 400, 400
_T = _M * _U * _V
_NW = 32
_C = 1024


def _adapter_body(x_hbm, m_hbm, u_hbm, v_hbm, ct_hbm, out_hbm,
                  u_v, v_v, m_v, x_v,
                  i11, i21, i12, i22,
                  w11, w21, w12, w22,
                  r11, r21, r12, r22,
                  out_v, sem):
    wid = lax.axis_index("s") * 2 + lax.axis_index("c")
    n_total = m_hbm.shape[0]
    per_w = n_total // _NW
    n_chunks = per_w // _C
    iota = lax.iota(jnp.int32, 16)

    def chunk_body(c, carry):
        off = wid * per_w + c * _C
        in_copies = [
            pltpu.async_copy(u_hbm.at[pl.ds(off, _C)], u_v, sem),
            pltpu.async_copy(v_hbm.at[pl.ds(off, _C)], v_v, sem),
            pltpu.async_copy(m_hbm.at[pl.ds(off, _C)], m_v, sem),
            pltpu.async_copy(x_hbm.at[pl.ds(3 * off, 3 * _C)], x_v, sem),
        ]
        for cp in in_copies:
            cp.wait()

        def prep(g, carry2):
            s = pl.multiple_of(g * 16, 16)
            u16 = u_v[pl.ds(s, 16)]
            v16 = v_v[pl.ds(s, 16)]
            m16 = m_v[pl.ds(s, 16)]
            iu = u16 * jnp.float32(_U)
            iu = jnp.where(iu == jnp.float32(_U), jnp.float32(_U - 1), iu)
            iv = v16 * jnp.float32(_V)
            iv = jnp.where(iv == jnp.float32(_V), jnp.float32(_V - 1), iv)
            i1 = iu.astype(jnp.int32)
            j1 = iv.astype(jnp.int32)
            ir = iu - i1.astype(jnp.float32)
            jr = iv - j1.astype(jnp.float32)
            i2 = jnp.where(i1 == _U - 1, 0, i1 + 1)
            j2 = jnp.where(j1 == _V - 1, 0, j1 + 1)
            base = m16 * (_U * _V)
            row1 = base + i1 * _V
            row2 = base + i2 * _V
            i11[pl.ds(s, 16)] = row1 + j1
            i21[pl.ds(s, 16)] = row2 + j1
            i12[pl.ds(s, 16)] = row1 + j2
            i22[pl.ds(s, 16)] = row2 + j2
            omi = 1.0 - ir
            omj = 1.0 - jr
            w11[pl.ds(s, 16)] = omi * omj
            w21[pl.ds(s, 16)] = ir * omj
            w12[pl.ds(s, 16)] = omi * jr
            w22[pl.ds(s, 16)] = ir * jr
            return carry2
        lax.fori_loop(0, _C // 16, prep, 0)

        copies = [pltpu.async_copy(ct_hbm.at[idx_v], rm_v, sem)
                  for idx_v, rm_v in ((i11, r11), (i21, r21),
                                      (i12, r12), (i22, r22))]
        for cp in copies:
            cp.wait()

        def comb(g, carry2):
            s = pl.multiple_of(g * 16, 16)
            rows = iota + s
            r3 = rows * 3
            wa = w11[pl.ds(s, 16)]
            wb = w21[pl.ds(s, 16)]
            wc = w12[pl.ds(s, 16)]
            wd = w22[pl.ds(s, 16)]

            def gat(ref, k):
                return plsc.load_gather(ref, [rows, jnp.full((16,), k, jnp.int32)])

            A = [wa * gat(r11, k) + wb * gat(r21, k)
                 + wc * gat(r12, k) + wd * gat(r22, k) for k in range(12)]
            x0 = plsc.load_gather(x_v, [r3])
            x1 = plsc.load_gather(x_v, [r3 + 1])
            x2 = plsc.load_gather(x_v, [r3 + 2])
            for j in range(3):
                yj = x0 * A[j] + x1 * A[3 + j] + x2 * A[6 + j] + A[9 + j]
                plsc.store_scatter(out_v, [r3 + j], yj)
            return carry2
        lax.fori_loop(0, _C // 16, comb, 0)

        pltpu.sync_copy(out_v, out_hbm.at[pl.ds(3 * off, 3 * _C)])
        return carry
    lax.fori_loop(0, n_chunks, chunk_body, 0)


def kernel(x, m, u, v, m_param, b_param):
    n = x.shape[0]
    xf = x.reshape(-1)
    ct = jnp.concatenate(
        [m_param.reshape(_T, 9), b_param.reshape(_T, 3),
         jnp.zeros((_T, 4), jnp.float32)], axis=1)
    f32, i32 = jnp.float32, jnp.int32
    mesh = plsc.VectorSubcoreMesh(core_axis_name="c", subcore_axis_name="s")
    f = pl.kernel(
        _adapter_body,
        mesh=mesh,
        out_type=jax.ShapeDtypeStruct((n * 3,), f32),
        compiler_params=pltpu.CompilerParams(
            needs_layout_passes=False, use_tc_tiling_on_sc=False),
        scratch_types=[
            pltpu.VMEM((_C,), f32),
            pltpu.VMEM((_C,), f32),
            pltpu.VMEM((_C,), i32),
            pltpu.VMEM((_C * 3,), f32),
            pltpu.VMEM((_C,), i32),
            pltpu.VMEM((_C,), i32),
            pltpu.VMEM((_C,), i32),
            pltpu.VMEM((_C,), i32),
            pltpu.VMEM((_C,), f32),
            pltpu.VMEM((_C,), f32),
            pltpu.VMEM((_C,), f32),
            pltpu.VMEM((_C,), f32),
            pltpu.VMEM((_C, 16), f32),
            pltpu.VMEM((_C, 16), f32),
            pltpu.VMEM((_C, 16), f32),
            pltpu.VMEM((_C, 16), f32),
            pltpu.VMEM((_C * 3,), f32),
            pltpu.SemaphoreType.DMA,
        ],
    )
    return f(xf, m, u, v, ct).reshape(n, 3)
